# baseline (speedup 1.0000x reference)
import numpy as np

import jax
import jax.numpy as jnp
from jax import lax
from jax.experimental import pallas as pl
from jax.experimental.pallas import tpu as pltpu
from jax.experimental.pallas import tpu_sc as plsc

N = 10000
E = 320000
D = 128
DP = D // 2

NC = 2
NS = 16
NW = NC * NS
C = 64
EP = 327680
TOTCH = EP // C
NCHT = TOTCH // NS
NB = 2
NQ = 8
NP = 10240
NP2 = NP // NC
RPT = NP // NS
RPH = NP2 // NS
ZR = 16
LG = D // 32

_PERM = np.concatenate(
    [32 * j + np.concatenate([np.arange(0, 32, 2), np.arange(1, 32, 2)])
     for j in range(LG)])


def _sc_body(pk_hbm, val_hbm, x_hbm, out_hbm, pkb, vbb, bufs, scaled, zbuf,
             xsh, hsh, sems, psems):
    c = lax.axis_index("c")
    s = lax.axis_index("s")
    base = c * NP2

    for k in range(RPT // C):
        pltpu.sync_copy(x_hbm.at[pl.ds(s * RPT + k * C, C)], bufs[0])
        pltpu.sync_copy(bufs[0], xsh.at[pl.ds(s * RPT + k * C, C)])

    def zrow(i, _):
        for j in range(D // 16):
            zbuf[i, pl.ds(j * 16, 16)] = jnp.zeros((16,), jnp.float32)
        return 0
    lax.fori_loop(0, ZR, zrow, 0)
    for k in range(RPH // ZR):
        pltpu.sync_copy(zbuf, hsh.at[pl.ds(s * RPH + k * ZR, ZR)])
    plsc.subcore_barrier()

    def pk_copy(t, q):
        return pltpu.make_async_copy(pk_hbm.at[s * NCHT + t], pkb[q],
                                     psems[q])

    def vb_copy(t, q):
        return pltpu.make_async_copy(
            val_hbm.at[pl.ds((s * NCHT + t) * C, C)], vbb[q], psems[q])

    def rows_copy(t, b, q):
        return pltpu.make_async_copy(xsh.at[pkb[q].at[1]], bufs[b], sems[b])

    def retarget(q):
        pk = pkb[q]
        for g in range(C // 16):
            d16 = pk[0, pl.ds(g * 16, 16)]
            ok = (d16 >= base) & (d16 < base + NP2)
            pk[0, pl.ds(g * 16, 16)] = jnp.where(ok, d16 - base, NP2)

    def process(t, b, q):
        rows_copy(t, b, q).wait()
        buf = bufs[b]
        vb = vbb[q]

        def scale(eb, _):
            v16 = vb[pl.ds(eb * 16, 16)]
            for i in range(16):
                e = eb * 16 + i
                sp = jnp.full((16,), v16[i], jnp.float32)
                for j in range(LG):
                    h = buf[e, pl.ds(j * 16, 16)]
                    lo = plsc.bitcast(h << 16, jnp.float32)
                    hi = plsc.bitcast(h & jnp.int32(-65536), jnp.float32)
                    scaled[e, pl.ds(j * 32, 16)] = lo * sp
                    scaled[e, pl.ds(j * 32 + 16, 16)] = hi * sp
            return 0
        lax.fori_loop(0, C // 16, scale, 0)
        pltpu.sync_copy(scaled, hsh.at[pkb[q].at[0]], add=True)

    for t in range(NQ):
        pk_copy(t, t).start()
        vb_copy(t, t).start()
    for t in range(NB):
        pk_copy(t, t).wait()
        vb_copy(t, t).wait()
        retarget(t)
        rows_copy(t, t, t).start()

    def octet(g, _):
        for u in range(NQ):
            t = NQ * g + u
            b = u % NB
            q = u % NQ
            process(t, b, q)

            @pl.when(t + NQ < NCHT)
            def _():
                pk_copy(t + NQ, q).start()
                vb_copy(t + NQ, q).start()

            @pl.when(t + NB < NCHT)
            def _():
                pk_copy(t + NB, (u + NB) % NQ).wait()
                vb_copy(t + NB, (u + NB) % NQ).wait()
                retarget((u + NB) % NQ)
                rows_copy(t + NB, b, (u + NB) % NQ).start()
        return 0
    lax.fori_loop(0, NCHT // NQ, octet, 0)

    plsc.subcore_barrier()
    pltpu.sync_copy(hsh.at[pl.ds(s * RPH, RPH)],
                    out_hbm.at[c, pl.ds(s * RPH, RPH)])


def _sc_entry(pk_hbm, val_hbm, x_hbm, out_hbm,
              pk0, pk1, pk2, pk3, pk4, pk5, pk6, pk7,
              vb0, vb1, vb2, vb3, vb4, vb5, vb6, vb7,
              rows0, rows1, scaled, zbuf, xsh, hsh,
              sem0, sem1,
              psem0, psem1, psem2, psem3, psem4, psem5, psem6, psem7):
    _sc_body(pk_hbm, val_hbm, x_hbm, out_hbm,
             (pk0, pk1, pk2, pk3, pk4, pk5, pk6, pk7),
             (vb0, vb1, vb2, vb3, vb4, vb5, vb6, vb7),
             (rows0, rows1), scaled, zbuf, xsh, hsh,
             (sem0, sem1),
             (psem0, psem1, psem2, psem3, psem4, psem5, psem6, psem7))


def _sc_scatter(pk, val, xpk):
    mesh = plsc.VectorSubcoreMesh(core_axis_name="c", subcore_axis_name="s")
    f = pl.kernel(
        _sc_entry,
        out_type=jax.ShapeDtypeStruct((NC, NP2, D), jnp.float32),
        mesh=mesh,
        compiler_params=pltpu.CompilerParams(needs_layout_passes=False,
                                             use_tc_tiling_on_sc=False),
        scratch_types=(
            [pltpu.VMEM((2, C), jnp.int32)] * NQ
            + [pltpu.VMEM((C,), jnp.float32)] * NQ
            + [pltpu.VMEM((C, DP), jnp.int32)] * NB
            + [pltpu.VMEM((C, D), jnp.float32),
               pltpu.VMEM((ZR, D), jnp.float32),
               pltpu.VMEM_SHARED((NP, DP), jnp.int32),
               pltpu.VMEM_SHARED((NP2 + 8, D), jnp.float32)]
            + [pltpu.SemaphoreType.DMA] * (NB + NQ)
        ),
    )
    return f(pk, val, xpk)


def _tc_body(hp_ref, wt_ref, b_ref, o_ref):
    y = jnp.dot(hp_ref[...], wt_ref[...], preferred_element_type=jnp.float32)
    o_ref[...] = jnp.maximum(y + b_ref[...], 0.0)


def _tc_linear(hp, wt, b):
    R = 2048
    grid = (NP // R,)
    return pl.pallas_call(
        _tc_body,
        grid=grid,
        in_specs=[
            pl.BlockSpec((R, D), lambda i: (i, 0)),
            pl.BlockSpec((D, D), lambda i: (0, 0)),
            pl.BlockSpec((1, D), lambda i: (0, 0)),
        ],
        out_specs=pl.BlockSpec((R, D), lambda i: (i, 0)),
        out_shape=jax.ShapeDtypeStruct((NP, D), jnp.float32),
    )(hp, wt, b)


def kernel(A_indices, A_values, X, W, b):
    pad = EP - E
    dst = jnp.pad(A_indices[0], (0, pad)).reshape(TOTCH, 1, C)
    src = jnp.pad(A_indices[1], (0, pad)).reshape(TOTCH, 1, C)
    val = jnp.pad(A_values, (0, pad))
    pk = jnp.concatenate([dst, src], axis=1)
    xbf = X.astype(jnp.bfloat16)
    xpk = lax.bitcast_convert_type(xbf.reshape(N, DP, 2), jnp.int32)
    xpk = jnp.pad(xpk, ((0, NP - N), (0, 0)))
    wtp = W.T[_PERM]
    partials = _sc_scatter(pk, val, xpk)
    return _tc_linear(partials.reshape(NP, D), wtp, b.reshape(1, D))[:N]


# speedup vs baseline: 1.8514x; 1.8514x over previous
import numpy as np

import jax
import jax.numpy as jnp
from jax import lax
from jax.experimental import pallas as pl
from jax.experimental.pallas import tpu as pltpu
from jax.experimental.pallas import tpu_sc as plsc

N = 10000
E = 320000
D = 128

NC = 2
NS = 16
NW = NC * NS
C = 128
EP = 327680
EPW = EP // NW
NCHUNK = EPW // C
NB = 2
NQ = 8
NP = 10240
RPT = NP // NS
ZR = 16
LG = D // 32

_PERM = np.concatenate(
    [32 * j + np.concatenate([np.arange(0, 32, 2), np.arange(1, 32, 2)])
     for j in range(LG)])


def _sc_body(pk_hbm, val_hbm, x_hbm, out_hbm, pkb, vbb, bufs, scaled, zbuf,
             hsh, sems, psems):
    c = lax.axis_index("c")
    s = lax.axis_index("s")
    w = c * NS + s

    def zrow(i, _):
        for j in range(D // 16):
            zbuf[i, pl.ds(j * 16, 16)] = jnp.zeros((16,), jnp.float32)
        return 0
    lax.fori_loop(0, ZR, zrow, 0)
    for k in range(RPT // ZR):
        pltpu.sync_copy(zbuf, hsh.at[pl.ds(s * RPT + k * ZR, ZR)])
    plsc.subcore_barrier()

    def pk_copy(t, q):
        return pltpu.make_async_copy(pk_hbm.at[w * NCHUNK + t], pkb[q],
                                     psems[q])

    def vb_copy(t, q):
        return pltpu.make_async_copy(
            val_hbm.at[pl.ds((w * NCHUNK + t) * C, C)], vbb[q], psems[q])

    def rows_copy(t, b, q):
        return pltpu.make_async_copy(x_hbm.at[pkb[q].at[1]], bufs[b], sems[b])

    def process(t, b, q):
        rows_copy(t, b, q).wait()
        buf = bufs[b]
        vb = vbb[q]

        def scale(eb, _):
            v16 = vb[pl.ds(eb * 16, 16)]
            for i in range(16):
                e = eb * 16 + i
                sp = jnp.full((16,), v16[i], jnp.float32)
                for j in range(LG):
                    h = buf[e, pl.ds(j * 16, 16)]
                    lo = plsc.bitcast(h << 16, jnp.float32)
                    hi = plsc.bitcast(h & jnp.int32(-65536), jnp.float32)
                    scaled[e, pl.ds(j * 32, 16)] = lo * sp
                    scaled[e, pl.ds(j * 32 + 16, 16)] = hi * sp
            return 0
        lax.fori_loop(0, C // 16, scale, 0)
        pltpu.sync_copy(scaled, hsh.at[pkb[q].at[0]], add=True)

    for t in range(NQ):
        pk_copy(t, t).start()
        vb_copy(t, t).start()
    for t in range(NB):
        pk_copy(t, t).wait()
        vb_copy(t, t).wait()
        rows_copy(t, t, t).start()

    def octet(g, _):
        for u in range(NQ):
            t = NQ * g + u
            b = u % NB
            q = u % NQ
            process(t, b, q)

            @pl.when(t + NQ < NCHUNK)
            def _():
                pk_copy(t + NQ, q).start()
                vb_copy(t + NQ, q).start()

            @pl.when(t + NB < NCHUNK)
            def _():
                pk_copy(t + NB, (u + NB) % NQ).wait()
                vb_copy(t + NB, (u + NB) % NQ).wait()
                rows_copy(t + NB, b, (u + NB) % NQ).start()
        return 0
    lax.fori_loop(0, NCHUNK // NQ, octet, 0)

    plsc.subcore_barrier()
    pltpu.sync_copy(hsh.at[pl.ds(s * RPT, RPT)],
                    out_hbm.at[c, pl.ds(s * RPT, RPT)])


def _sc_entry(pk_hbm, val_hbm, x_hbm, out_hbm,
              pk0, pk1, pk2, pk3, pk4, pk5, pk6, pk7,
              vb0, vb1, vb2, vb3, vb4, vb5, vb6, vb7,
              rows0, rows1, scaled, zbuf, hsh,
              sem0, sem1,
              psem0, psem1, psem2, psem3, psem4, psem5, psem6, psem7):
    _sc_body(pk_hbm, val_hbm, x_hbm, out_hbm,
             (pk0, pk1, pk2, pk3, pk4, pk5, pk6, pk7),
             (vb0, vb1, vb2, vb3, vb4, vb5, vb6, vb7),
             (rows0, rows1), scaled, zbuf, hsh,
             (sem0, sem1),
             (psem0, psem1, psem2, psem3, psem4, psem5, psem6, psem7))


def _sc_scatter(pk, val, xbf):
    mesh = plsc.VectorSubcoreMesh(core_axis_name="c", subcore_axis_name="s")
    f = pl.kernel(
        _sc_entry,
        out_type=jax.ShapeDtypeStruct((NC, NP, D), jnp.float32),
        mesh=mesh,
        compiler_params=pltpu.CompilerParams(needs_layout_passes=False,
                                             use_tc_tiling_on_sc=False),
        scratch_types=(
            [pltpu.VMEM((2, C), jnp.int32)] * NQ
            + [pltpu.VMEM((C,), jnp.float32)] * NQ
            + [pltpu.VMEM((C, D // 2), jnp.int32)] * NB
            + [pltpu.VMEM((C, D), jnp.float32),
               pltpu.VMEM((ZR, D), jnp.float32),
               pltpu.VMEM_SHARED((NP, D), jnp.float32)]
            + [pltpu.SemaphoreType.DMA] * (NB + NQ)
        ),
    )
    return f(pk, val, xbf)


def _tc_body(hp_ref, wt_ref, b_ref, o_ref):
    h = hp_ref[0] + hp_ref[1]
    y = jnp.dot(h, wt_ref[...], preferred_element_type=jnp.float32)
    o_ref[...] = jnp.maximum(y + b_ref[...], 0.0)


def _tc_linear(partials, wt, b):
    R = 2048
    grid = (NP // R,)
    return pl.pallas_call(
        _tc_body,
        grid=grid,
        in_specs=[
            pl.BlockSpec((NC, R, D), lambda i: (0, i, 0)),
            pl.BlockSpec((D, D), lambda i: (0, 0)),
            pl.BlockSpec((1, D), lambda i: (0, 0)),
        ],
        out_specs=pl.BlockSpec((R, D), lambda i: (i, 0)),
        out_shape=jax.ShapeDtypeStruct((NP, D), jnp.float32),
    )(partials, wt, b)


def kernel(A_indices, A_values, X, W, b):
    pad = EP - E
    dst = jnp.pad(A_indices[0], (0, pad)).reshape(NW * NCHUNK, 1, C)
    src = jnp.pad(A_indices[1], (0, pad)).reshape(NW * NCHUNK, 1, C)
    val = jnp.pad(A_values, (0, pad))
    pk = jnp.concatenate([dst, src], axis=1)
    xbf = X.astype(jnp.bfloat16)
    xpk = lax.bitcast_convert_type(xbf.reshape(N, D // 2, 2), jnp.int32)
    wtp = W.T[_PERM]
    partials = _sc_scatter(pk, val, xpk)
    return _tc_linear(partials, wtp, b.reshape(1, D))[:N]
